# Initial kernel scaffold; baseline (speedup 1.0000x reference)
import jax
import jax.numpy as jnp
from jax.experimental import pallas as pl


def kernel(x, edge_index, W1, b1, W2, b2, Wfc, bfc):
    raise NotImplementedError("write your pallas kernel here")


# speedup vs baseline: 7.8585x; 7.8585x over previous
import functools

import jax
import jax.numpy as jnp
from jax import lax
from jax.experimental import pallas as pl
from jax.experimental.pallas import tpu as pltpu
from jax.experimental.pallas import tpu_sc as plsc

N = 100000
E = 1600000
NC, NS = 2, 16
NW = NC * NS
BLK = 128
NP = 100096
NCH = NP // NS
F = 8

EW = E // NW
NB_W = EW // BLK
TW_W = EW - NB_W * BLK
ET = E // NS
NB_T = ET // BLK
TW_T = ET - NB_T * BLK

RB = 2000
GRID = N // RB


def _mesh():
    return plsc.VectorSubcoreMesh(core_axis_name="c", subcore_axis_name="s")


_SC_PARAMS = pltpu.CompilerParams(use_tc_tiling_on_sc=False)


def _sc_degree(dst):
    @functools.partial(
        pl.kernel,
        out_type=jax.ShapeDtypeStruct((NC * NP, F), jnp.float32),
        mesh=_mesh(),
        compiler_params=_SC_PARAMS,
        scratch_types=[
            pltpu.VMEM_SHARED((NP, F), jnp.float32),
            pltpu.VMEM((NCH, F), jnp.float32),
            pltpu.VMEM((BLK, F), jnp.float32),
            pltpu.VMEM((TW_W, F), jnp.float32),
            pltpu.VMEM((BLK,), jnp.int32),
            pltpu.VMEM((TW_W,), jnp.int32),
        ],
    )
    def deg_kernel(dst_hbm, zeros_hbm, ones_hbm, deg2_hbm,
                   deg_sh, stage, ones_v, ones_t, didx, didx_t):
        c = lax.axis_index("c")
        s = lax.axis_index("s")
        wid = s * NC + c
        pltpu.sync_copy(zeros_hbm.at[pl.ds(s * NCH, NCH)], stage)
        pltpu.sync_copy(stage, deg_sh.at[pl.ds(s * NCH, NCH)])
        pltpu.sync_copy(ones_hbm, ones_v)
        pltpu.sync_copy(ones_hbm.at[pl.ds(0, TW_W)], ones_t)
        plsc.subcore_barrier()
        base = wid * EW

        @pl.loop(0, NB_W)
        def _(j):
            pltpu.sync_copy(dst_hbm.at[pl.ds(base + j * BLK, BLK)], didx)
            pltpu.sync_copy(ones_v, deg_sh.at[didx], add=True)

        pltpu.sync_copy(dst_hbm.at[pl.ds(base + NB_W * BLK, TW_W)], didx_t)
        pltpu.sync_copy(ones_t, deg_sh.at[didx_t], add=True)
        plsc.subcore_barrier()
        pltpu.sync_copy(deg_sh.at[pl.ds(s * NCH, NCH)], stage)
        pltpu.sync_copy(stage, deg2_hbm.at[pl.ds(c * NP + s * NCH, NCH)])

    zeros, ones = lax.optimization_barrier(
        (jnp.zeros((NP, F), jnp.float32), jnp.ones((BLK, F), jnp.float32)))
    return deg_kernel(dst, zeros, ones)


def _sc_agg8(src, dst, p1):
    @functools.partial(
        pl.kernel,
        out_type=jax.ShapeDtypeStruct((NC * NP, F), jnp.float32),
        mesh=_mesh(),
        compiler_params=_SC_PARAMS,
        scratch_types=[
            pltpu.VMEM_SHARED((NP, F), jnp.float32),
            pltpu.VMEM((NCH, F), jnp.float32),
            pltpu.VMEM((BLK,), jnp.int32),
            pltpu.VMEM((BLK,), jnp.int32),
            pltpu.VMEM((BLK, F), jnp.float32),
            pltpu.VMEM((TW_W,), jnp.int32),
            pltpu.VMEM((TW_W,), jnp.int32),
            pltpu.VMEM((TW_W, F), jnp.float32),
            pltpu.SemaphoreType.DMA,
        ],
    )
    def agg_kernel(src_hbm, dst_hbm, p1_hbm, zeros_hbm, u1_hbm,
                   agg_sh, stage, sidx, didx, rows, sidx_t, didx_t, rows_t,
                   sem):
        c = lax.axis_index("c")
        s = lax.axis_index("s")
        wid = s * NC + c
        pltpu.sync_copy(zeros_hbm.at[pl.ds(s * NCH, NCH)], stage)
        pltpu.sync_copy(stage, agg_sh.at[pl.ds(s * NCH, NCH)])
        plsc.subcore_barrier()
        base = wid * EW

        @pl.loop(0, NB_W)
        def _(j):
            off = base + j * BLK
            pltpu.sync_copy(src_hbm.at[pl.ds(off, BLK)], sidx)
            pltpu.async_copy(p1_hbm.at[sidx], rows, sem).wait()
            pltpu.sync_copy(dst_hbm.at[pl.ds(off, BLK)], didx)
            pltpu.sync_copy(rows, agg_sh.at[didx], add=True)

        off = base + NB_W * BLK
        pltpu.sync_copy(src_hbm.at[pl.ds(off, TW_W)], sidx_t)
        pltpu.async_copy(p1_hbm.at[sidx_t], rows_t, sem).wait()
        pltpu.sync_copy(dst_hbm.at[pl.ds(off, TW_W)], didx_t)
        pltpu.sync_copy(rows_t, agg_sh.at[didx_t], add=True)
        plsc.subcore_barrier()
        pltpu.sync_copy(agg_sh.at[pl.ds(s * NCH, NCH)], stage)
        pltpu.sync_copy(stage, u1_hbm.at[pl.ds(c * NP + s * NCH, NCH)])

    zeros = lax.optimization_barrier(jnp.zeros((NP, F), jnp.float32))
    return agg_kernel(src, dst, p1, zeros)


def _sc_agg8x4(src, dst, p2a, p2b, p2c, p2d):
    @functools.partial(
        pl.kernel,
        out_type=jax.ShapeDtypeStruct((4 * NP, F), jnp.float32),
        mesh=_mesh(),
        compiler_params=_SC_PARAMS,
        scratch_types=[
            pltpu.VMEM_SHARED((NP, F), jnp.float32),
            pltpu.VMEM((NCH, F), jnp.float32),
            pltpu.VMEM((BLK,), jnp.int32),
            pltpu.VMEM((BLK,), jnp.int32),
            pltpu.VMEM((BLK, F), jnp.float32),
            pltpu.VMEM((TW_T,), jnp.int32),
            pltpu.VMEM((TW_T,), jnp.int32),
            pltpu.VMEM((TW_T, F), jnp.float32),
            pltpu.SemaphoreType.DMA,
        ],
    )
    def agg_kernel(src_hbm, dst_hbm, p2a_hbm, p2b_hbm, p2c_hbm, p2d_hbm,
                   zeros_hbm, u2_hbm,
                   agg_sh, stage, sidx, didx, rows, sidx_t, didx_t, rows_t,
                   sem):
        c = lax.axis_index("c")
        s = lax.axis_index("s")
        base = s * ET

        def round_(tbl_hbm, r):
            q = 2 * c + r
            pltpu.sync_copy(zeros_hbm.at[pl.ds(s * NCH, NCH)], stage)
            pltpu.sync_copy(stage, agg_sh.at[pl.ds(s * NCH, NCH)])
            plsc.subcore_barrier()

            @pl.loop(0, NB_T)
            def _(j):
                off = base + j * BLK
                pltpu.sync_copy(src_hbm.at[pl.ds(off, BLK)], sidx)
                pltpu.async_copy(tbl_hbm.at[sidx], rows, sem).wait()
                pltpu.sync_copy(dst_hbm.at[pl.ds(off, BLK)], didx)
                pltpu.sync_copy(rows, agg_sh.at[didx], add=True)

            off = base + NB_T * BLK
            pltpu.sync_copy(src_hbm.at[pl.ds(off, TW_T)], sidx_t)
            pltpu.async_copy(tbl_hbm.at[sidx_t], rows_t, sem).wait()
            pltpu.sync_copy(dst_hbm.at[pl.ds(off, TW_T)], didx_t)
            pltpu.sync_copy(rows_t, agg_sh.at[didx_t], add=True)
            plsc.subcore_barrier()
            pltpu.sync_copy(agg_sh.at[pl.ds(s * NCH, NCH)], stage)
            pltpu.sync_copy(stage, u2_hbm.at[pl.ds(q * NP + s * NCH, NCH)])

        @pl.when(c == 0)
        def _():
            round_(p2a_hbm, 0)
            round_(p2b_hbm, 1)

        @pl.when(c == 1)
        def _():
            round_(p2c_hbm, 0)
            round_(p2d_hbm, 1)

    zeros = lax.optimization_barrier(jnp.zeros((NP, F), jnp.float32))
    return agg_kernel(src, dst, p2a, p2b, p2c, p2d, zeros)


def _tc_prep(deg2, x):
    def body(deg2_ref, x_ref, p1_ref):
        deg = deg2_ref[:, 0] + deg2_ref[:, 1] + 1.0
        dis = lax.rsqrt(deg)
        z = jnp.zeros((RB, 4), jnp.float32)
        p1_ref[...] = jnp.concatenate(
            [dis[:, None] * x_ref[...], dis[:, None], z], axis=1)

    return pl.pallas_call(
        body,
        grid=(GRID,),
        in_specs=[pl.BlockSpec((RB, 2), lambda i: (i, 0)),
                  pl.BlockSpec((RB, 3), lambda i: (i, 0))],
        out_specs=pl.BlockSpec((RB, F), lambda i: (i, 0)),
        out_shape=jax.ShapeDtypeStruct((N, F), jnp.float32),
    )(deg2, x)


def _tc_layer1(deg2, x, u1, W1, b1):
    def body(deg2_ref, x_ref, u1_ref, W1_ref, b1_ref,
             h_ref, p2a_ref, p2b_ref, p2c_ref, p2d_ref):
        deg = deg2_ref[:, 0] + deg2_ref[:, 1] + 1.0
        dis = lax.rsqrt(deg)
        dinv = 1.0 / deg
        u = u1_ref[0, :, 0:3] + u1_ref[1, :, 0:3]
        pre = dis[:, None] * u + dinv[:, None] * x_ref[...]
        h = jnp.dot(pre, W1_ref[...], preferred_element_type=jnp.float32)
        h = jnp.maximum(h + b1_ref[...], 0.0)
        h_ref[...] = h
        p2 = dis[:, None] * h
        p2a_ref[...] = p2[:, 0:8]
        p2b_ref[...] = p2[:, 8:16]
        p2c_ref[...] = p2[:, 16:24]
        p2d_ref[...] = p2[:, 24:32]

    return pl.pallas_call(
        body,
        grid=(GRID,),
        in_specs=[pl.BlockSpec((RB, 2), lambda i: (i, 0)),
                  pl.BlockSpec((RB, 3), lambda i: (i, 0)),
                  pl.BlockSpec((2, RB, F), lambda i: (0, i, 0)),
                  pl.BlockSpec((3, 32), lambda i: (0, 0)),
                  pl.BlockSpec((1, 32), lambda i: (0, 0))],
        out_specs=[pl.BlockSpec((RB, 32), lambda i: (i, 0)),
                   pl.BlockSpec((RB, F), lambda i: (i, 0)),
                   pl.BlockSpec((RB, F), lambda i: (i, 0)),
                   pl.BlockSpec((RB, F), lambda i: (i, 0)),
                   pl.BlockSpec((RB, F), lambda i: (i, 0))],
        out_shape=[jax.ShapeDtypeStruct((N, 32), jnp.float32),
                   jax.ShapeDtypeStruct((N, F), jnp.float32),
                   jax.ShapeDtypeStruct((N, F), jnp.float32),
                   jax.ShapeDtypeStruct((N, F), jnp.float32),
                   jax.ShapeDtypeStruct((N, F), jnp.float32)],
    )(deg2, x, u1, W1, b1)


def _tc_layer2(deg2, h, u2, W2, b2, Wfc, bfc):
    def body(deg2_ref, h_ref, u2_ref, W2_ref, b2_ref, Wfc_ref, bfc_ref,
             y_ref, acc_ref):
        i = pl.program_id(0)
        deg = deg2_ref[:, 0] + deg2_ref[:, 1] + 1.0
        dis = lax.rsqrt(deg)
        dinv = 1.0 / deg
        u2f = jnp.concatenate(
            [u2_ref[0], u2_ref[1], u2_ref[2], u2_ref[3]], axis=1)
        pre = dis[:, None] * u2f + dinv[:, None] * h_ref[...]
        hh = jnp.dot(pre, W2_ref[...], preferred_element_type=jnp.float32)
        hh = jnp.maximum(hh + b2_ref[...], 0.0)
        part = jnp.sum(hh, axis=0, keepdims=True)

        @pl.when(i == 0)
        def _():
            acc_ref[...] = jnp.zeros_like(acc_ref)

        acc_ref[...] += part

        @pl.when(i == GRID - 1)
        def _():
            g = acc_ref[...]
            z = jnp.sum(g[0, :] * Wfc_ref[:, 0]) + bfc_ref[0, 0]
            y_ref[...] = jnp.reshape(1.0 / (1.0 + jnp.exp(-z)), (1, 1))

    return pl.pallas_call(
        body,
        grid=(GRID,),
        in_specs=[pl.BlockSpec((RB, 2), lambda i: (i, 0)),
                  pl.BlockSpec((RB, 32), lambda i: (i, 0)),
                  pl.BlockSpec((4, RB, F), lambda i: (0, i, 0)),
                  pl.BlockSpec((32, 64), lambda i: (0, 0)),
                  pl.BlockSpec((1, 64), lambda i: (0, 0)),
                  pl.BlockSpec((64, 1), lambda i: (0, 0)),
                  pl.BlockSpec((1, 1), lambda i: (0, 0))],
        out_specs=pl.BlockSpec((1, 1), lambda i: (0, 0)),
        out_shape=jax.ShapeDtypeStruct((1, 1), jnp.float32),
        scratch_shapes=[pltpu.VMEM((1, 64), jnp.float32)],
    )(deg2, h, u2, W2, b2, Wfc, bfc)


def kernel(x, edge_index, W1, b1, W2, b2, Wfc, bfc):
    src = edge_index[0].astype(jnp.int32)
    dst = edge_index[1].astype(jnp.int32)
    deg2 = _sc_degree(dst).reshape(NC, NP, F)[:, :N, 0].T
    p1 = _tc_prep(deg2, x)
    u1 = _sc_agg8(src, dst, p1).reshape(NC, NP, F)[:, :N, :]
    h, p2a, p2b, p2c, p2d = _tc_layer1(deg2, x, u1, W1, b1.reshape(1, 32))
    u2 = _sc_agg8x4(src, dst, p2a, p2b, p2c, p2d).reshape(4, NP, F)[:, :N, :]
    yy = _tc_layer2(deg2, h, u2, W2, b2.reshape(1, 64), Wfc,
                    bfc.reshape(1, 1))
    return yy.reshape((1,))
